# baseline (speedup 1.0000x reference)
import jax
import jax.numpy as jnp
from jax.experimental import pallas as pl

_EMB = 64
_DIN = 32
_NROWS = 4
_FOLD = 4


def _fused_kernel(seg_ref, x_ref, w_ref, b_ref, table_ref, o_ref):
    x = x_ref[...]
    w = w_ref[...]
    bias = b_ref[...]
    table = table_ref[...]
    seg = seg_ref[...]

    rvec = jax.lax.broadcasted_iota(
        jnp.int32, (1, _FOLD * _NROWS), 1) % _NROWS
    onehot = (seg == rvec).astype(jnp.float32)
    emb = jnp.dot(onehot, table, preferred_element_type=jnp.float32)

    dense = jax.lax.dot_general(
        x, w, (((2,), (0,)), ((), ())),
        preferred_element_type=jnp.float32)
    res = dense + (emb + bias)[None, :, :]
    p4 = res.shape[1]
    for k in range(_FOLD):
        o_ref[:, pl.Slice(k, p4, _FOLD), :] = (
            res[:, :, k * _EMB:(k + 1) * _EMB])


@jax.jit
def kernel(x, W, b, table, seg):
    B, P, DIN = x.shape
    P4 = P // _FOLD
    BT = 8

    x4 = x.reshape(B, P4, _FOLD * DIN)
    eye = jnp.eye(_FOLD, dtype=jnp.float32)
    wbig = jnp.kron(eye, W)
    tbig = jnp.kron(eye, table)
    b4 = jnp.tile(b, _FOLD).reshape(1, _FOLD * _EMB)
    segrep = jnp.repeat(seg.reshape(P4, _FOLD), _NROWS, axis=1)

    grid = (B // BT,)
    return pl.pallas_call(
        _fused_kernel,
        grid=grid,
        in_specs=[
            pl.BlockSpec((P4, _FOLD * _NROWS), lambda i: (0, 0)),
            pl.BlockSpec((BT, P4, _FOLD * DIN), lambda i: (i, 0, 0)),
            pl.BlockSpec((_FOLD * DIN, _FOLD * _EMB), lambda i: (0, 0)),
            pl.BlockSpec((1, _FOLD * _EMB), lambda i: (0, 0)),
            pl.BlockSpec((_FOLD * _NROWS, _FOLD * _EMB), lambda i: (0, 0)),
        ],
        out_specs=pl.BlockSpec((BT, P, _EMB), lambda i: (i, 0, 0)),
        out_shape=jax.ShapeDtypeStruct((B, P, _EMB), jnp.float32),
    )(segrep, x4, wbig, b4, tbig)


# speedup vs baseline: 5.3216x; 5.3216x over previous
import jax
import jax.numpy as jnp
from jax.experimental import pallas as pl

_EMB = 64
_DIN = 32
_K = _DIN + 8


def _fused_kernel(seg_ref, x_ref, lhs_ref, o_ref):
    x = x_ref[...]
    lhs = lhs_ref[...]
    seg = seg_ref[...]

    pt = x.shape[0]
    bsz = x.shape[2]
    onehot = (seg == jax.lax.broadcasted_iota(
        jnp.int32, (pt, _K - _DIN, bsz), 1)).astype(jnp.float32)
    rhs = jnp.concatenate([x, onehot], axis=1)
    lhsb = jnp.broadcast_to(lhs[None], (pt, _EMB, _K))
    o_ref[...] = jax.lax.dot_general(
        lhsb, rhs, (((2,), (1,)), ((0,), (0,))),
        preferred_element_type=jnp.float32)


@jax.jit
def kernel(x, W, b, table, seg):
    B, P, DIN = x.shape
    PT = 32

    xt = jnp.transpose(x, (1, 2, 0))
    tb = table + b[None, :]
    lhs = jnp.concatenate(
        [W.T, tb.T, jnp.zeros((_EMB, _K - _DIN - table.shape[0]),
                              jnp.float32)], axis=1)
    seg3 = seg.reshape(P, 1, 1)

    grid = (P // PT,)
    out_t = pl.pallas_call(
        _fused_kernel,
        grid=grid,
        in_specs=[
            pl.BlockSpec((PT, 1, 1), lambda i: (i, 0, 0)),
            pl.BlockSpec((PT, DIN, B), lambda i: (i, 0, 0)),
            pl.BlockSpec((_EMB, _K), lambda i: (0, 0)),
        ],
        out_specs=pl.BlockSpec((PT, _EMB, B), lambda i: (i, 0, 0)),
        out_shape=jax.ShapeDtypeStruct((P, _EMB, B), jnp.float32),
    )(seg3, xt, lhs)
    return jnp.transpose(out_t, (2, 0, 1))
